# baseline (speedup 1.0000x reference)
import jax
import jax.numpy as jnp
from jax.experimental import pallas as pl
import jax.experimental.pallas.tpu as pltpu

N = 2048
IN_DIM = 128
OUT_DIM = 16
BLOCK_D = 256


def _gat_kernel(attn_b_ref, adj_ref, h_ref, fc_w_ref, fc_b_ref, attn_w_ref,
                out_ref, z_ref, zt_ref, alpha_ref):
    j = pl.program_id(0)

    @pl.when(j == 0)
    def _init():
        z = jax.lax.dot_general(
            h_ref[...], fc_w_ref[...],
            dimension_numbers=(((1,), (1,)), ((), ())),
            preferred_element_type=jnp.float32,
        ) + fc_b_ref[...]
        z_ref[...] = z
        zt_ref[...] = z.T
        w_src_rep = jnp.broadcast_to(attn_w_ref[:, :OUT_DIM], (BLOCK_D, OUT_DIM))
        alpha_ref[...] = jax.lax.dot_general(
            z, w_src_rep,
            dimension_numbers=(((1,), (1,)), ((), ())),
            preferred_element_type=jnp.float32,
        )

    zt = zt_ref[...]
    zt_d = zt_ref[:, pl.ds(j * BLOCK_D, BLOCK_D)]
    beta = jax.lax.dot_general(
        attn_w_ref[:, OUT_DIM:], zt_d,
        dimension_numbers=(((1,), (0,)), ((), ())),
        preferred_element_type=jnp.float32,
    ) + attn_b_ref[0, 0]
    t = alpha_ref[...] + beta
    e = jnp.maximum(t, 0.01 * t) * adj_ref[...]
    out_ref[...] = jax.lax.dot_general(
        zt, e,
        dimension_numbers=(((1,), (0,)), ((), ())),
        preferred_element_type=jnp.float32,
    )


def kernel(adj, h, fc_w, fc_b, attn_w, attn_b):
    fc_b2 = fc_b.reshape(1, OUT_DIM)
    attn_b2 = attn_b.reshape(1, 1)
    grid = (N // BLOCK_D,)
    out_t = pl.pallas_call(
        _gat_kernel,
        grid=grid,
        in_specs=[
            pl.BlockSpec(memory_space=pltpu.SMEM),
            pl.BlockSpec((N, BLOCK_D), lambda j: (0, j)),
            pl.BlockSpec((N, IN_DIM), lambda j: (0, 0)),
            pl.BlockSpec((OUT_DIM, IN_DIM), lambda j: (0, 0)),
            pl.BlockSpec((1, OUT_DIM), lambda j: (0, 0)),
            pl.BlockSpec((1, 2 * OUT_DIM), lambda j: (0, 0)),
        ],
        out_specs=pl.BlockSpec((OUT_DIM, BLOCK_D), lambda j: (0, j)),
        out_shape=jax.ShapeDtypeStruct((OUT_DIM, N), jnp.float32),
        scratch_shapes=[
            pltpu.VMEM((N, OUT_DIM), jnp.float32),
            pltpu.VMEM((OUT_DIM, N), jnp.float32),
            pltpu.VMEM((N, BLOCK_D), jnp.float32),
        ],
    )(attn_b2, adj, h, fc_w, fc_b2, attn_w)
    return out_t.T


# speedup vs baseline: 1.3241x; 1.3241x over previous
import jax
import jax.numpy as jnp
from jax.experimental import pallas as pl
import jax.experimental.pallas.tpu as pltpu

N = 2048
IN_DIM = 128
OUT_DIM = 16
BLOCK_D = 1024


def _gat_kernel(attn_b_ref, adj_ref, h_ref, fc_w_ref, fc_b_ref, attn_w_ref,
                out_ref, z_ref, zt_ref, alpha_ref):
    j = pl.program_id(0)

    @pl.when(j == 0)
    def _init():
        z = jax.lax.dot_general(
            h_ref[...], fc_w_ref[...],
            dimension_numbers=(((1,), (1,)), ((), ())),
            preferred_element_type=jnp.float32,
        ) + fc_b_ref[...]
        z_ref[...] = z
        zt_ref[...] = z.T
        w_src_rep = jnp.broadcast_to(attn_w_ref[:, :OUT_DIM], (BLOCK_D, OUT_DIM))
        alpha_ref[...] = jax.lax.dot_general(
            z, w_src_rep,
            dimension_numbers=(((1,), (1,)), ((), ())),
            preferred_element_type=jnp.float32,
        )

    zt = zt_ref[...]
    zt_d = zt_ref[:, pl.ds(j * BLOCK_D, BLOCK_D)]
    beta = jax.lax.dot_general(
        attn_w_ref[:, OUT_DIM:], zt_d,
        dimension_numbers=(((1,), (0,)), ((), ())),
        preferred_element_type=jnp.float32,
    ) + attn_b_ref[0, 0]
    t = alpha_ref[...] + beta
    e = jnp.maximum(t, 0.01 * t) * adj_ref[...]
    out_ref[...] = jax.lax.dot_general(
        zt, e,
        dimension_numbers=(((1,), (0,)), ((), ())),
        preferred_element_type=jnp.float32,
    )


def kernel(adj, h, fc_w, fc_b, attn_w, attn_b):
    fc_b2 = fc_b.reshape(1, OUT_DIM)
    attn_b2 = attn_b.reshape(1, 1)
    grid = (N // BLOCK_D,)
    out_t = pl.pallas_call(
        _gat_kernel,
        grid=grid,
        in_specs=[
            pl.BlockSpec(memory_space=pltpu.SMEM),
            pl.BlockSpec((N, BLOCK_D), lambda j: (0, j)),
            pl.BlockSpec((N, IN_DIM), lambda j: (0, 0)),
            pl.BlockSpec((OUT_DIM, IN_DIM), lambda j: (0, 0)),
            pl.BlockSpec((1, OUT_DIM), lambda j: (0, 0)),
            pl.BlockSpec((1, 2 * OUT_DIM), lambda j: (0, 0)),
        ],
        out_specs=pl.BlockSpec((OUT_DIM, BLOCK_D), lambda j: (0, j)),
        out_shape=jax.ShapeDtypeStruct((OUT_DIM, N), jnp.float32),
        scratch_shapes=[
            pltpu.VMEM((N, OUT_DIM), jnp.float32),
            pltpu.VMEM((OUT_DIM, N), jnp.float32),
            pltpu.VMEM((N, BLOCK_D), jnp.float32),
        ],
    )(attn_b2, adj, h, fc_w, fc_b2, attn_w)
    return out_t.T
